# baseline (speedup 1.0000x reference)
import jax
import jax.numpy as jnp
from jax.experimental import pallas as pl
from jax.experimental.pallas import tpu as pltpu

_NSLOT = 3


def _window_kernel(t8, g3, out, vbuf, gbuf, big_sem, g_sem, in_sem, out_sem):
    for b in range(4):
        pltpu.make_async_copy(
            t8.at[b, pl.ds(256, 768)], out.at[b, pl.ds(256, 768)], big_sem.at[b]
        ).start()

    g_cp = pltpu.make_async_copy(g3, gbuf, g_sem)
    g_cp.start()

    def in_cp(b, slot):
        return pltpu.make_async_copy(
            t8.at[b, pl.ds(0, 256)], vbuf.at[slot], in_sem.at[slot]
        )

    def out_cp(b, slot):
        return pltpu.make_async_copy(
            vbuf.at[slot], out.at[b, pl.ds(0, 256)], out_sem.at[slot]
        )

    for b in range(_NSLOT):
        in_cp(b, b).start()
    g_cp.wait()
    for b in range(4):
        slot = b % _NSLOT
        in_cp(b, slot).wait()
        for h in range(12):
            vbuf[slot, 1:256, h, 1:256] += gbuf[b, h, :, :]
        out_cp(b, slot).start()
        nxt = b + _NSLOT
        if nxt < 4:
            out_cp(b, slot).wait()
            in_cp(nxt, slot).start()
    for b in range(4 - _NSLOT, 4):
        out_cp(b, b % _NSLOT).wait()
    for b in range(4):
        pltpu.make_async_copy(
            t8.at[b, pl.ds(256, 768)], out.at[b, pl.ds(256, 768)], big_sem.at[b]
        ).wait()


def kernel(transpose_8, getitem_3, view_4):
    del view_4
    g3 = getitem_3.reshape(4, 12, 255, 255)
    out = pl.pallas_call(
        _window_kernel,
        in_specs=[
            pl.BlockSpec(memory_space=pltpu.MemorySpace.HBM),
            pl.BlockSpec(memory_space=pltpu.MemorySpace.HBM),
        ],
        out_specs=pl.BlockSpec(memory_space=pltpu.MemorySpace.HBM),
        out_shape=jax.ShapeDtypeStruct((4, 1024, 12, 513), transpose_8.dtype),
        scratch_shapes=[
            pltpu.VMEM((_NSLOT, 256, 12, 513), jnp.float32),
            pltpu.VMEM((4, 12, 255, 255), jnp.float32),
            pltpu.SemaphoreType.DMA((4,)),
            pltpu.SemaphoreType.DMA,
            pltpu.SemaphoreType.DMA((_NSLOT,)),
            pltpu.SemaphoreType.DMA((_NSLOT,)),
        ],
    )(transpose_8, g3)
    return (out,)


# speedup vs baseline: 7.1762x; 7.1762x over previous
import functools
import jax
import jax.numpy as jnp
from jax import lax
from jax.experimental import pallas as pl
from jax.experimental.pallas import tpu as pltpu, tpu_sc as plsc

_CHUNK = 2
_NCHUNK = 64
_NPAIR = 16


def _make_sc_kernel():
    mesh = plsc.VectorSubcoreMesh(core_axis_name="c", subcore_axis_name="s")

    @functools.partial(
        pl.kernel,
        mesh=mesh,
        out_type=jax.ShapeDtypeStruct((4, 1024, 12, 513), jnp.float32),
        scratch_types=[
            pltpu.VMEM((_CHUNK, 12, 513), jnp.float32),
            pltpu.VMEM((_CHUNK, 12, 513), jnp.float32),
            pltpu.VMEM((12, 16, 255), jnp.float32),
            pltpu.SemaphoreType.DMA,
            pltpu.SemaphoreType.DMA,
            pltpu.SemaphoreType.DMA,
            pltpu.SemaphoreType.DMA,
            pltpu.SemaphoreType.DMA,
        ],
    )
    def k(t8_hbm, g3_hbm, g3tail_hbm, out_hbm, vbuf0, vbuf1, gbuf,
          isem0, isem1, osem0, osem1, gsem):
        wid = lax.axis_index("s") * 2 + lax.axis_index("c")
        b = wid // 8
        jt = wid % 8
        r0 = jt * 128
        is_win = jt < 2
        bufs = (vbuf0, vbuf1)
        isems = (isem0, isem1)
        osems = (osem0, osem1)

        def in_cp(i, slot):
            return pltpu.make_async_copy(
                t8_hbm.at[b, pl.ds(r0 + i * _CHUNK, _CHUNK)], bufs[slot],
                isems[slot])

        def out_cp(i, slot):
            return pltpu.make_async_copy(
                bufs[slot], out_hbm.at[b, pl.ds(r0 + i * _CHUNK, _CHUNK)],
                osems[slot])

        def stage_norm(srow, off):
            cps = [
                pltpu.make_async_copy(
                    g3_hbm.at[b * 12 + h, pl.ds(srow, 8)],
                    gbuf.at[h, pl.ds(off, 8)], gsem)
                for h in range(12)
            ]
            for c in cps:
                c.start()
            for c in cps:
                c.wait()

        def stage_tail():
            cps = [
                pltpu.make_async_copy(
                    g3tail_hbm.at[b * 12 + h], gbuf.at[h, pl.ds(8, 8)], gsem)
                for h in range(12)
            ]
            for c in cps:
                c.start()
            for c in cps:
                c.wait()

        lane = lax.iota(jnp.int32, 16)
        k0_mask = lane > 0
        shift_idx = jnp.maximum(lane - 1, 0)

        def window_add(i, slot, is_tailpair):
            t0 = r0 + i * _CHUNK
            s_lo = jnp.where(t0 == 0, 1, 0)
            vb = bufs[slot]

            def s_body(s, _):
                t = t0 + s
                r = t - 1
                grow = jnp.where(is_tailpair,
                                 8 + (t - 248),
                                 8 * ((r // 8) % 2) + r % 8)
                def h_body(h, __):
                    a = gbuf[h, grow, pl.ds(0, 16)]
                    gv = a.at[shift_idx].get(mode="promise_in_bounds")
                    vb[s, h, pl.ds(0, 16)] += jnp.where(k0_mask, gv, 0.0)
                    for g in range(1, 16):
                        vb[s, h, pl.ds(16 * g, 16)] += gbuf[h, grow, pl.ds(16 * g - 1, 16)]
                    return __

                return lax.fori_loop(0, 12, h_body, _)

            lax.fori_loop(s_lo, _CHUNK, s_body, 0)

        in_cp(0, 0).start()

        @pl.when(is_win & (jt == 1))
        def _():
            stage_norm(pl.multiple_of(r0 - 8, 8), 8)

        def body(p, carry):
            is_tailpair = is_win & (jt == 1) & (p == _NPAIR - 1)
            is_normpair = is_win & jnp.logical_not((jt == 1) & (p == _NPAIR - 1))

            @pl.when(is_normpair)
            def _():
                stage_norm(pl.multiple_of(r0 + 8 * p, 8),
                           pl.multiple_of(8 * (p % 2), 8))

            @pl.when(is_tailpair)
            def _():
                stage_tail()

            for c in range(4):
                i = 4 * p + c
                slot = c % 2
                in_cp(i, slot).wait()

                @pl.when(is_win)
                def _():
                    window_add(i, slot, is_tailpair)

                @pl.when(i >= 1)
                def _():
                    out_cp(i - 1, 1 - slot).wait()

                @pl.when(i + 1 < _NCHUNK)
                def _():
                    in_cp(i + 1, 1 - slot).start()

                out_cp(i, slot).start()
            return carry

        lax.fori_loop(0, _NPAIR, body, 0)
        out_cp(_NCHUNK - 1, 1).wait()

    return k


_sc_kernel = _make_sc_kernel()


def kernel(transpose_8, getitem_3, view_4):
    del view_4
    g3_tail = getitem_3[:, 247:, :]
    return (_sc_kernel(transpose_8, getitem_3, g3_tail),)


# speedup vs baseline: 7.2223x; 1.0064x over previous
import functools
import jax
import jax.numpy as jnp
from jax import lax
from jax.experimental import pallas as pl
from jax.experimental.pallas import tpu as pltpu, tpu_sc as plsc

_CHUNK = 4
_NCHUNK = 32


def _make_sc_kernel():
    mesh = plsc.VectorSubcoreMesh(core_axis_name="c", subcore_axis_name="s")

    @functools.partial(
        pl.kernel,
        mesh=mesh,
        out_type=jax.ShapeDtypeStruct((4, 1024, 12, 513), jnp.float32),
        scratch_types=[
            pltpu.VMEM((_CHUNK, 12, 513), jnp.float32),
            pltpu.VMEM((_CHUNK, 12, 513), jnp.float32),
            pltpu.VMEM((12, 8, 255), jnp.float32),
            pltpu.SemaphoreType.DMA,
            pltpu.SemaphoreType.DMA,
            pltpu.SemaphoreType.DMA,
            pltpu.SemaphoreType.DMA,
            pltpu.SemaphoreType.DMA,
        ],
    )
    def k(t8_hbm, g3_hbm, g3tail_hbm, out_hbm, vbuf0, vbuf1, gbuf,
          isem0, isem1, osem0, osem1, gsem):
        wid = lax.axis_index("s") * 2 + lax.axis_index("c")
        b = wid // 8
        jt = wid % 8
        r0 = jt * 128
        is_win = jt < 2
        bufs = (vbuf0, vbuf1)
        isems = (isem0, isem1)
        osems = (osem0, osem1)

        def in_cp(i, slot):
            return pltpu.make_async_copy(
                t8_hbm.at[b, pl.ds(r0 + i * _CHUNK, _CHUNK)], bufs[slot],
                isems[slot])

        def out_cp(i, slot):
            return pltpu.make_async_copy(
                bufs[slot], out_hbm.at[b, pl.ds(r0 + i * _CHUNK, _CHUNK)],
                osems[slot])

        def stage_norm(srow):
            cps = [
                pltpu.make_async_copy(
                    g3_hbm.at[b * 12 + h, pl.ds(srow, 8)], gbuf.at[h], gsem)
                for h in range(12)
            ]
            for c in cps:
                c.start()
            for c in cps:
                c.wait()

        def stage_tail():
            cps = [
                pltpu.make_async_copy(
                    g3tail_hbm.at[b * 12 + h], gbuf.at[h], gsem)
                for h in range(12)
            ]
            for c in cps:
                c.start()
            for c in cps:
                c.wait()

        lane = lax.iota(jnp.int32, 16)
        k0_mask = lane > 0
        shift_idx = jnp.maximum(lane - 1, 0)

        def add_token(vb, s, grow):
            def h_body(h, __):
                a = gbuf[h, grow, pl.ds(0, 16)]
                gv = a.at[shift_idx].get(mode="promise_in_bounds")
                vb[s, h, pl.ds(0, 16)] += jnp.where(k0_mask, gv, 0.0)
                for g in range(1, 16):
                    vb[s, h, pl.ds(16 * g, 16)] += gbuf[h, grow, pl.ds(16 * g - 1, 16)]
                return __

            return lax.fori_loop(0, 12, h_body, 0)

        in_cp(0, 0).start()

        @pl.when(is_win & (jt == 1))
        def _():
            stage_norm(pl.multiple_of(r0 - 8, 8))

        def body(p2, carry):
            for c in range(4):
                i = 4 * p2 + c
                slot = c % 2
                p = 2 * p2 + c // 2
                first = (c % 2) == 0
                t0 = r0 + i * _CHUNK
                is_tail = is_win & (jt == 1) & (p == 15)
                src0 = jnp.where(is_tail, 247, r0 + 8 * p)

                in_cp(i, slot).wait()
                vb = bufs[slot]

                if first:
                    @pl.when(is_win & (t0 > 0))
                    def _():
                        add_token(vb, 0, 7)

                    @pl.when(is_win & jnp.logical_not(is_tail))
                    def _():
                        stage_norm(pl.multiple_of(r0 + 8 * p, 8))

                    @pl.when(is_tail)
                    def _():
                        stage_tail()

                    @pl.when(is_win)
                    def _():
                        lax.fori_loop(
                            1, _CHUNK,
                            lambda s, _: add_token(vb, s, t0 + s - 1 - src0), 0)
                else:
                    @pl.when(is_win)
                    def _():
                        lax.fori_loop(
                            0, _CHUNK,
                            lambda s, _: add_token(vb, s, t0 + s - 1 - src0), 0)

                @pl.when(i >= 1)
                def _():
                    out_cp(i - 1, 1 - slot).wait()

                @pl.when(i + 1 < _NCHUNK)
                def _():
                    in_cp(i + 1, 1 - slot).start()

                out_cp(i, slot).start()
            return carry

        lax.fori_loop(0, _NCHUNK // 4, body, 0)
        out_cp(_NCHUNK - 1, 1).wait()

    return k


_sc_kernel = _make_sc_kernel()


def kernel(transpose_8, getitem_3, view_4):
    del view_4
    g3_tail = getitem_3[:, 247:, :]
    return (_sc_kernel(transpose_8, getitem_3, g3_tail),)


# speedup vs baseline: 9.3471x; 1.2942x over previous
import functools
import jax
import jax.numpy as jnp
from jax import lax
from jax.experimental import pallas as pl
from jax.experimental.pallas import tpu as pltpu, tpu_sc as plsc

_CHUNK = 4
_NCHUNK = 32


def _make_sc_kernel():
    mesh = plsc.VectorSubcoreMesh(core_axis_name="c", subcore_axis_name="s")

    @functools.partial(
        pl.kernel,
        mesh=mesh,
        out_type=jax.ShapeDtypeStruct((4, 1024, 12, 513), jnp.float32),
        scratch_types=[
            pltpu.VMEM((_CHUNK, 12, 513), jnp.float32),
            pltpu.VMEM((_CHUNK, 12, 513), jnp.float32),
            pltpu.VMEM((12, 8, 255), jnp.float32),
            pltpu.SemaphoreType.DMA,
            pltpu.SemaphoreType.DMA,
            pltpu.SemaphoreType.DMA,
            pltpu.SemaphoreType.DMA,
            pltpu.SemaphoreType.DMA,
        ],
    )
    def k(t8_hbm, g3_hbm, g3tail_hbm, out_hbm, vbuf0, vbuf1, gbuf,
          isem0, isem1, osem0, osem1, gsem):
        wid = lax.axis_index("s") * 2 + lax.axis_index("c")
        b = wid // 8
        jt = wid % 8
        w0 = 32 * jt
        bufs = (vbuf0, vbuf1)
        isems = (isem0, isem1)
        osems = (osem0, osem1)

        def row_of(i):
            p2 = i // 8
            c = i % 8
            return jnp.where(c < 6,
                             256 + 96 * jt + _CHUNK * (6 * p2 + c),
                             w0 + 8 * p2 + _CHUNK * (c - 6))

        def in_cp(i, slot):
            return pltpu.make_async_copy(
                t8_hbm.at[b, pl.ds(row_of(i), _CHUNK)], bufs[slot],
                isems[slot])

        def out_cp(i, slot):
            return pltpu.make_async_copy(
                bufs[slot], out_hbm.at[b, pl.ds(row_of(i), _CHUNK)],
                osems[slot])

        def stage_norm(srow):
            cps = [
                pltpu.make_async_copy(
                    g3_hbm.at[b * 12 + h, pl.ds(srow, 8)], gbuf.at[h], gsem)
                for h in range(12)
            ]
            for c in cps:
                c.start()
            for c in cps:
                c.wait()

        def stage_tail():
            cps = [
                pltpu.make_async_copy(
                    g3tail_hbm.at[b * 12 + h], gbuf.at[h], gsem)
                for h in range(12)
            ]
            for c in cps:
                c.start()
            for c in cps:
                c.wait()

        lane = lax.iota(jnp.int32, 16)
        k0_mask = lane > 0
        shift_idx = jnp.maximum(lane - 1, 0)

        def add_token(vb, s, grow):
            def h_body(h, __):
                a = gbuf[h, grow, pl.ds(0, 16)]
                gv = a.at[shift_idx].get(mode="promise_in_bounds")
                vb[s, h, pl.ds(0, 16)] += jnp.where(k0_mask, gv, 0.0)
                for g in range(1, 16):
                    vb[s, h, pl.ds(16 * g, 16)] += gbuf[h, grow, pl.ds(16 * g - 1, 16)]
                return __

            return lax.fori_loop(0, 12, h_body, 0)

        in_cp(0, 0).start()

        @pl.when(jt > 0)
        def _():
            stage_norm(pl.multiple_of(w0 - 8, 8))

        def body(p2, carry):
            T = w0 + 8 * p2
            is_tail = T == 248
            src0 = jnp.where(is_tail, 247, T)
            for c in range(8):
                i = 8 * p2 + c
                slot = c % 2
                in_cp(i, slot).wait()
                vb = bufs[slot]

                if c == 6:
                    @pl.when(T > 0)
                    def _():
                        add_token(vb, 0, 7)

                    @pl.when(jnp.logical_not(is_tail))
                    def _():
                        stage_norm(pl.multiple_of(T, 8))

                    @pl.when(is_tail)
                    def _():
                        stage_tail()

                    lax.fori_loop(
                        1, _CHUNK,
                        lambda s, _: add_token(vb, s, T + s - 1 - src0), 0)
                elif c == 7:
                    lax.fori_loop(
                        0, _CHUNK,
                        lambda s, _: add_token(vb, s, T + 4 + s - 1 - src0), 0)

                @pl.when(i >= 1)
                def _():
                    out_cp(i - 1, 1 - slot).wait()

                @pl.when(i + 1 < _NCHUNK)
                def _():
                    in_cp(i + 1, 1 - slot).start()

                out_cp(i, slot).start()
            return carry

        lax.fori_loop(0, _NCHUNK // 8, body, 0)
        out_cp(_NCHUNK - 1, 1).wait()

    return k


_sc_kernel = _make_sc_kernel()


def kernel(transpose_8, getitem_3, view_4):
    del view_4
    g3_tail = getitem_3[:, 247:, :]
    return (_sc_kernel(transpose_8, getitem_3, g3_tail),)
